# baseline (speedup 1.0000x reference)
import functools
import math

import jax
import jax.numpy as jnp
from jax.experimental import pallas as pl
from jax.experimental.pallas import tpu as pltpu

GS = 128
HEADS = 16
DH = 64
CROSS = math.log(0.125)
BETA_LF = 0.675
BETA_MF = 0.225
BETA_HF = 0.1
MQ = 256
NSUB = MQ // GS


def _attend(qt, ks, kn, vs, vn, ql, ln):
    k = jnp.concatenate([ks, kn], axis=0)
    v = jnp.concatenate([vs, vn], axis=0)
    kl = jnp.concatenate([ql, ln], axis=1)
    bias = jnp.where(ql.T == kl, 0.0, CROSS)
    outs = []
    for h in range(HEADS):
        sl = slice(h * DH, (h + 1) * DH)
        lg = jax.lax.dot_general(qt[:, sl], k[:, sl], (((1,), (1,)), ((), ())),
                                 preferred_element_type=jnp.float32)
        p = jnp.exp(lg + bias)
        s = jnp.sum(p, axis=-1, keepdims=True)
        o = jnp.dot(p.astype(jnp.bfloat16), v[:, sl],
                    preferred_element_type=jnp.float32)
        outs.append(o / s)
    return jnp.concatenate(outs, axis=1)


def _level_kernel(s, fuse, *refs):
    if s == 1:
        (x_ref, lt_ref, w_ref), rest = refs[:3], refs[3:]
    else:
        (x_ref, srow_ref, pm_ref, st_ref, lt_ref, w_ref), rest = \
            refs[:6], refs[6:]
    if fuse:
        (amf_ref, alf_ref, xres_ref, wp_ref, out_ref,
         kc_ref, vc_ref, lc_ref) = rest
    else:
        out_ref, kc_ref, vc_ref, lc_ref = rest
    j = pl.program_id(1)
    C = x_ref.shape[-1]

    xb = x_ref[0]
    if s == 1:
        px = xb
        plab = lt_ref[0]
    else:
        R = GS * s
        w = jnp.clip(srow_ref[0], 1e-6, None)
        sg = st_ref[0]
        lg_ = lt_ref[0]
        if s == 2:
            plab = jnp.where(sg[0:1] >= sg[1:2], lg_[0:1], lg_[1:2])
        else:
            m01 = jnp.maximum(sg[0:1], sg[1:2])
            l01 = jnp.where(sg[0:1] >= sg[1:2], lg_[0:1], lg_[1:2])
            m23 = jnp.maximum(sg[2:3], sg[3:4])
            l23 = jnp.where(sg[2:3] >= sg[3:4], lg_[2:3], lg_[3:4])
            plab = jnp.where(m01 >= m23, l01, l23)
        subs = []
        for t in range(NSUB):
            wt = w[:, t * R:(t + 1) * R]
            wm = pm_ref[:] * jnp.broadcast_to(wt, (GS, R))
            den = jnp.sum(wm, axis=1, keepdims=True)
            wn = (wm / den).astype(jnp.bfloat16)
            subs.append(jnp.dot(wn, xb[t * R:(t + 1) * R, :],
                                preferred_element_type=jnp.float32
                                ).astype(jnp.bfloat16))
        px = jnp.concatenate(subs, axis=0)

    qkv = jnp.dot(px, w_ref[:],
                  preferred_element_type=jnp.float32).astype(jnp.bfloat16)
    q = qkv[:, :C]
    k = qkv[:, C:2 * C]
    v = qkv[:, 2 * C:]

    k_carry = jnp.where(j == 0, k[(NSUB - 1) * GS:], kc_ref[:])
    v_carry = jnp.where(j == 0, v[(NSUB - 1) * GS:], vc_ref[:])
    l_carry = jnp.where(j == 0, plab[:, (NSUB - 1) * GS:], lc_ref[:])

    a_subs = []
    for t in range(NSUB):
        row = slice(t * GS, (t + 1) * GS)
        ql = plab[:, row]
        if t < NSUB - 1:
            nrow = slice((t + 1) * GS, (t + 2) * GS)
            a_subs.append(_attend(q[row], k[row], k[nrow], v[row], v[nrow],
                                  ql, plab[:, nrow]))
        else:
            a_subs.append(_attend(q[row], k[row], k_carry, v[row], v_carry,
                                  ql, l_carry))
    a = jnp.concatenate(a_subs, axis=0)

    kc_ref[:] = k[:GS]
    vc_ref[:] = v[:GS]
    lc_ref[:] = plab[:, :GS]

    if fuse:
        amf = amf_ref[0].astype(jnp.float32)
        up2 = jnp.broadcast_to(amf[:, None, :], (MQ // 2, 2, C)).reshape(MQ, C)
        alf = alf_ref[0].astype(jnp.float32)
        up4 = jnp.broadcast_to(alf[:, None, :], (MQ // 4, 4, C)).reshape(MQ, C)
        fused = BETA_HF * a + BETA_MF * up2 + BETA_LF * up4
        out_ref[0] = jnp.dot(fused.astype(jnp.bfloat16), wp_ref[:],
                             preferred_element_type=jnp.float32) + xres_ref[0]
    else:
        out_ref[0] = a.astype(jnp.bfloat16)


def _run_level(s, x_bf, scores, labels, wqkv, fuse_args, interpret=False):
    B, N, C = x_bf.shape
    np_ = N // s
    ngq = np_ // MQ
    R = MQ * s
    rev = lambda j: ngq - 1 - j
    in_specs = [pl.BlockSpec((1, R, C), lambda b, j: (b, rev(j), 0))]
    args = [x_bf]
    if s > 1:
        srow = scores.reshape(B * ngq, 1, R)
        st = scores.reshape(B * ngq, MQ, s).transpose(0, 2, 1)
        pmask = (jnp.arange(GS * s)[None, :] // s == jnp.arange(GS)[:, None]
                 ).astype(jnp.float32)
        in_specs += [
            pl.BlockSpec((1, 1, R), lambda b, j, g=ngq: (b * g + rev(j), 0, 0)),
            pl.BlockSpec((GS, GS * s), lambda b, j: (0, 0)),
            pl.BlockSpec((1, s, MQ), lambda b, j, g=ngq: (b * g + rev(j), 0, 0)),
        ]
        args += [srow, pmask, st]
    lt = labels.reshape(B * ngq, MQ, s).transpose(0, 2, 1)
    in_specs += [
        pl.BlockSpec((1, s, MQ), lambda b, j, g=ngq: (b * g + rev(j), 0, 0)),
        pl.BlockSpec((C, 3 * C), lambda b, j: (0, 0)),
    ]
    args += [lt, wqkv]
    if fuse_args is not None:
        amf, alf, x, wp = fuse_args
        in_specs += [
            pl.BlockSpec((1, MQ // 2, C), lambda b, j: (b, rev(j), 0)),
            pl.BlockSpec((1, MQ // 4, C), lambda b, j: (b, rev(j), 0)),
            pl.BlockSpec((1, MQ, C), lambda b, j: (b, rev(j), 0)),
            pl.BlockSpec((C, C), lambda b, j: (0, 0)),
        ]
        args += [amf, alf, x, wp]
    kern = functools.partial(_level_kernel, s, fuse_args is not None)
    out = pl.pallas_call(
        kern,
        grid=(B, ngq),
        in_specs=in_specs,
        out_specs=pl.BlockSpec((1, MQ, C), lambda b, j: (b, rev(j), 0)),
        out_shape=jax.ShapeDtypeStruct(
            (B, np_, C), jnp.float32 if fuse_args is not None else jnp.bfloat16),
        scratch_shapes=[
            pltpu.VMEM((GS, C), jnp.bfloat16),
            pltpu.VMEM((GS, C), jnp.bfloat16),
            pltpu.VMEM((1, GS), jnp.int32),
        ],
        interpret=interpret,
    )(*args)
    return out


def _impl(x, labels, scores, Wq_hf, Wk_hf, Wv_hf, Wq_mf, Wk_mf, Wv_mf,
          Wq_lf, Wk_lf, Wv_lf, Wp, interpret=False):
    B, N, C = x.shape
    labels = labels.astype(jnp.int32)
    x_bf = x.astype(jnp.bfloat16)
    scale = 1.0 / math.sqrt(DH)

    def wcat(wq, wk, wv):
        return jnp.concatenate([wq * scale, wk, wv], axis=1
                               ).astype(jnp.bfloat16)

    a_lf = _run_level(4, x_bf, scores, labels, wcat(Wq_lf, Wk_lf, Wv_lf),
                      None, interpret)
    a_mf = _run_level(2, x_bf, scores, labels, wcat(Wq_mf, Wk_mf, Wv_mf),
                      None, interpret)
    out = _run_level(1, x_bf, scores, labels, wcat(Wq_hf, Wk_hf, Wv_hf),
                     (a_mf, a_lf, x, Wp.astype(jnp.bfloat16)), interpret)
    return out


def kernel(x, labels, scores, Wq_hf, Wk_hf, Wv_hf, Wq_mf, Wk_mf, Wv_mf,
           Wq_lf, Wk_lf, Wv_lf, Wp):
    return _impl(x, labels, scores, Wq_hf, Wk_hf, Wv_hf, Wq_mf, Wk_mf,
                 Wv_mf, Wq_lf, Wk_lf, Wv_lf, Wp)


# speedup vs baseline: 1.0523x; 1.0523x over previous
import functools
import math

import jax
import jax.numpy as jnp
from jax.experimental import pallas as pl
from jax.experimental.pallas import tpu as pltpu

GS = 128
HEADS = 16
DH = 64
CROSS = math.log(0.125)
BETA_LF = 0.675
BETA_MF = 0.225
BETA_HF = 0.1
MQ = 512
NSUB = MQ // GS


def _attend(qt, ks, kn, vs, vn, ql, ln):
    k = jnp.concatenate([ks, kn], axis=0)
    v = jnp.concatenate([vs, vn], axis=0)
    kl = jnp.concatenate([ql, ln], axis=1)
    bias = jnp.where(ql.T == kl, 0.0, CROSS)
    outs = []
    for h in range(HEADS):
        sl = slice(h * DH, (h + 1) * DH)
        lg = jax.lax.dot_general(qt[:, sl], k[:, sl], (((1,), (1,)), ((), ())),
                                 preferred_element_type=jnp.float32)
        p = jnp.exp(lg + bias)
        s = jnp.sum(p, axis=-1, keepdims=True)
        o = jnp.dot(p.astype(jnp.bfloat16), v[:, sl],
                    preferred_element_type=jnp.float32)
        outs.append(o / s)
    return jnp.concatenate(outs, axis=1)


def _level_kernel(s, fuse, *refs):
    if s == 1:
        (x_ref, lt_ref, w_ref), rest = refs[:3], refs[3:]
    else:
        (x_ref, srow_ref, pm_ref, st_ref, lt_ref, w_ref), rest = \
            refs[:6], refs[6:]
    if fuse:
        (amf_ref, alf_ref, xres_ref, wp_ref, out_ref,
         kc_ref, vc_ref, lc_ref) = rest
    else:
        out_ref, kc_ref, vc_ref, lc_ref = rest
    j = pl.program_id(1)
    C = x_ref.shape[-1]

    xb = x_ref[0]
    if s == 1:
        px = xb
        plab = lt_ref[0]
    else:
        R = GS * s
        w = jnp.clip(srow_ref[0], 1e-6, None)
        sg = st_ref[0]
        lg_ = lt_ref[0]
        if s == 2:
            plab = jnp.where(sg[0:1] >= sg[1:2], lg_[0:1], lg_[1:2])
        else:
            m01 = jnp.maximum(sg[0:1], sg[1:2])
            l01 = jnp.where(sg[0:1] >= sg[1:2], lg_[0:1], lg_[1:2])
            m23 = jnp.maximum(sg[2:3], sg[3:4])
            l23 = jnp.where(sg[2:3] >= sg[3:4], lg_[2:3], lg_[3:4])
            plab = jnp.where(m01 >= m23, l01, l23)
        subs = []
        for t in range(NSUB):
            wt = w[:, t * R:(t + 1) * R]
            wm = pm_ref[:] * jnp.broadcast_to(wt, (GS, R))
            den = jnp.sum(wm, axis=1, keepdims=True)
            wn = (wm / den).astype(jnp.bfloat16)
            subs.append(jnp.dot(wn, xb[t * R:(t + 1) * R, :],
                                preferred_element_type=jnp.float32
                                ).astype(jnp.bfloat16))
        px = jnp.concatenate(subs, axis=0)

    qkv = jnp.dot(px, w_ref[:],
                  preferred_element_type=jnp.float32).astype(jnp.bfloat16)
    q = qkv[:, :C]
    k = qkv[:, C:2 * C]
    v = qkv[:, 2 * C:]

    k_carry = jnp.where(j == 0, k[(NSUB - 1) * GS:], kc_ref[:])
    v_carry = jnp.where(j == 0, v[(NSUB - 1) * GS:], vc_ref[:])
    l_carry = jnp.where(j == 0, plab[:, (NSUB - 1) * GS:], lc_ref[:])

    a_subs = []
    for t in range(NSUB):
        row = slice(t * GS, (t + 1) * GS)
        ql = plab[:, row]
        if t < NSUB - 1:
            nrow = slice((t + 1) * GS, (t + 2) * GS)
            a_subs.append(_attend(q[row], k[row], k[nrow], v[row], v[nrow],
                                  ql, plab[:, nrow]))
        else:
            a_subs.append(_attend(q[row], k[row], k_carry, v[row], v_carry,
                                  ql, l_carry))
    a = jnp.concatenate(a_subs, axis=0)

    kc_ref[:] = k[:GS]
    vc_ref[:] = v[:GS]
    lc_ref[:] = plab[:, :GS]

    if fuse:
        amf = amf_ref[0].astype(jnp.float32)
        up2 = jnp.broadcast_to(amf[:, None, :], (MQ // 2, 2, C)).reshape(MQ, C)
        alf = alf_ref[0].astype(jnp.float32)
        up4 = jnp.broadcast_to(alf[:, None, :], (MQ // 4, 4, C)).reshape(MQ, C)
        fused = BETA_HF * a + BETA_MF * up2 + BETA_LF * up4
        out_ref[0] = jnp.dot(fused.astype(jnp.bfloat16), wp_ref[:],
                             preferred_element_type=jnp.float32) + xres_ref[0]
    else:
        out_ref[0] = a.astype(jnp.bfloat16)


def _run_level(s, x_bf, scores, labels, wqkv, fuse_args, interpret=False):
    B, N, C = x_bf.shape
    np_ = N // s
    ngq = np_ // MQ
    R = MQ * s
    rev = lambda j: ngq - 1 - j
    in_specs = [pl.BlockSpec((1, R, C), lambda b, j: (b, rev(j), 0))]
    args = [x_bf]
    if s > 1:
        srow = scores.reshape(B * ngq, 1, R)
        st = scores.reshape(B * ngq, MQ, s).transpose(0, 2, 1)
        pmask = (jnp.arange(GS * s)[None, :] // s == jnp.arange(GS)[:, None]
                 ).astype(jnp.float32)
        in_specs += [
            pl.BlockSpec((1, 1, R), lambda b, j, g=ngq: (b * g + rev(j), 0, 0)),
            pl.BlockSpec((GS, GS * s), lambda b, j: (0, 0)),
            pl.BlockSpec((1, s, MQ), lambda b, j, g=ngq: (b * g + rev(j), 0, 0)),
        ]
        args += [srow, pmask, st]
    lt = labels.reshape(B * ngq, MQ, s).transpose(0, 2, 1)
    in_specs += [
        pl.BlockSpec((1, s, MQ), lambda b, j, g=ngq: (b * g + rev(j), 0, 0)),
        pl.BlockSpec((C, 3 * C), lambda b, j: (0, 0)),
    ]
    args += [lt, wqkv]
    if fuse_args is not None:
        amf, alf, x, wp = fuse_args
        in_specs += [
            pl.BlockSpec((1, MQ // 2, C), lambda b, j: (b, rev(j), 0)),
            pl.BlockSpec((1, MQ // 4, C), lambda b, j: (b, rev(j), 0)),
            pl.BlockSpec((1, MQ, C), lambda b, j: (b, rev(j), 0)),
            pl.BlockSpec((C, C), lambda b, j: (0, 0)),
        ]
        args += [amf, alf, x, wp]
    kern = functools.partial(_level_kernel, s, fuse_args is not None)
    out = pl.pallas_call(
        kern,
        grid=(B, ngq),
        in_specs=in_specs,
        out_specs=pl.BlockSpec((1, MQ, C), lambda b, j: (b, rev(j), 0)),
        out_shape=jax.ShapeDtypeStruct(
            (B, np_, C), jnp.float32 if fuse_args is not None else jnp.bfloat16),
        scratch_shapes=[
            pltpu.VMEM((GS, C), jnp.bfloat16),
            pltpu.VMEM((GS, C), jnp.bfloat16),
            pltpu.VMEM((1, GS), jnp.int32),
        ],
        interpret=interpret,
    )(*args)
    return out


def _impl(x, labels, scores, Wq_hf, Wk_hf, Wv_hf, Wq_mf, Wk_mf, Wv_mf,
          Wq_lf, Wk_lf, Wv_lf, Wp, interpret=False):
    B, N, C = x.shape
    labels = labels.astype(jnp.int32)
    x_bf = x.astype(jnp.bfloat16)
    scale = 1.0 / math.sqrt(DH)

    def wcat(wq, wk, wv):
        return jnp.concatenate([wq * scale, wk, wv], axis=1
                               ).astype(jnp.bfloat16)

    a_lf = _run_level(4, x_bf, scores, labels, wcat(Wq_lf, Wk_lf, Wv_lf),
                      None, interpret)
    a_mf = _run_level(2, x_bf, scores, labels, wcat(Wq_mf, Wk_mf, Wv_mf),
                      None, interpret)
    out = _run_level(1, x_bf, scores, labels, wcat(Wq_hf, Wk_hf, Wv_hf),
                     (a_mf, a_lf, x, Wp.astype(jnp.bfloat16)), interpret)
    return out


def kernel(x, labels, scores, Wq_hf, Wk_hf, Wv_hf, Wq_mf, Wk_mf, Wv_mf,
           Wq_lf, Wk_lf, Wv_lf, Wp):
    return _impl(x, labels, scores, Wq_hf, Wk_hf, Wv_hf, Wq_mf, Wk_mf,
                 Wv_mf, Wq_lf, Wk_lf, Wv_lf, Wp)
